# baseline (speedup 1.0000x reference)
import functools

import jax
import jax.numpy as jnp
from jax import lax
from jax.experimental import pallas as pl
from jax.experimental.pallas import tpu as pltpu
from jax.experimental.pallas import tpu_sc as plsc

SEQ_LEN = 2048
BT_SIZE = 4
D_MODEL = 4096

NC = 2
NS = 16
NW = NC * NS
STC = 8
NCHUNK = (SEQ_LEN // 8) // STC
L = 16


def _sc_body(x_hbm, p_hbm, z_hbm, pos_v, ibufs, obufs, in_sems, out_sems, psem):
    wid = lax.axis_index("s") * NC + lax.axis_index("c")

    iota = lax.iota(jnp.int32, L)
    zero_v = jnp.zeros((L,), jnp.int32)
    st_pat = lax.shift_right_logical(iota, 3)
    sl_pat = lax.bitwise_and(iota, jnp.full((L, ), 7, jnp.int32))

    def start_in(c, slot):
        pltpu.make_async_copy(
            x_hbm.at[:, pl.ds(c * STC, STC), pl.ds(wid, 1), :, :],
            ibufs.at[slot],
            in_sems.at[slot],
        ).start()

    def wait_in(c, slot):
        pltpu.make_async_copy(
            x_hbm.at[:, pl.ds(c * STC, STC), pl.ds(wid, 1), :, :],
            ibufs.at[slot],
            in_sems.at[slot],
        ).wait()

    def out_copy(c, qtr):
        t0 = lax.shift_right_logical(c, 1)
        q0 = lax.bitwise_and(c, 1) * 64 + qtr * 16
        return pltpu.make_async_copy(
            obufs.at[qtr],
            z_hbm.at[pl.ds(wid * 128, 128), pl.ds(t0, 1), :, pl.ds(q0, 16)],
            out_sems.at[qtr],
        )

    def transpose_quarter(slot, qtr):
        j0 = qtr * 16
        idx_st = st_pat + (j0 // 8)
        for b in range(BT_SIZE):
            idx_b = jnp.full((L,), b, jnp.int32)
            ln0 = jnp.full((L,), 0, jnp.int32)

            def body(i, idx_ln):
                cur = idx_ln
                row = i * 8
                for u in range(8):
                    vals = plsc.load_gather(
                        ibufs.at[slot],
                        [idx_b, idx_st, zero_v, sl_pat, cur],
                    )
                    obufs[qtr, row + u, 0, b, :] = vals
                    cur = cur + 1
                return cur

            lax.fori_loop(0, 16, body, ln0)

    @pl.when(wid == NW - 1)
    def _():
        pltpu.sync_copy(p_hbm, pos_v)
        for b in range(BT_SIZE):
            pltpu.make_async_copy(
                pos_v,
                z_hbm.at[pl.ds(D_MODEL, 1), :, pl.ds(b, 1), :],
                psem,
            ).start()
        for b in range(BT_SIZE):
            pltpu.make_async_copy(
                pos_v,
                z_hbm.at[pl.ds(D_MODEL, 1), :, pl.ds(b, 1), :],
                psem,
            ).wait()

    start_in(0, 0)
    start_in(1, 1)

    def step(g, carry):
        for slot in range(2):
            c = 2 * g + slot
            wait_in(c, slot)
            for qtr in range(4):
                if slot == 1:
                    out_copy(c - 1, qtr).wait()
                else:

                    @pl.when(g > 0)
                    def _():
                        out_copy(c - 1, qtr).wait()

                transpose_quarter(slot, qtr)
                out_copy(c, qtr).start()

            @pl.when(g < NCHUNK // 2 - 1)
            def _():
                start_in(c + 2, slot)

        return carry

    lax.fori_loop(0, NCHUNK // 2, step, 0)
    for qtr in range(4):
        out_copy(NCHUNK - 1, qtr).wait()


def kernel(inputs, pos_table):
    xv = inputs.reshape(BT_SIZE, 256, 8, 32, 128).transpose(0, 1, 3, 2, 4)
    pv = pos_table.reshape(1, 16, 1, 128)
    mesh = plsc.VectorSubcoreMesh(core_axis_name="c", subcore_axis_name="s")
    sc = functools.partial(
        pl.kernel,
        mesh=mesh,
        out_type=jax.ShapeDtypeStruct((D_MODEL + 1, 16, BT_SIZE, 128), jnp.float32),
        scratch_types=[
            pltpu.VMEM((1, 16, 1, 128), jnp.float32),
            pltpu.VMEM((2, BT_SIZE, STC, 1, 8, 128), jnp.float32),
            pltpu.VMEM((4, 128, 1, BT_SIZE, 16), jnp.float32),
            pltpu.SemaphoreType.DMA((2,)),
            pltpu.SemaphoreType.DMA((4,)),
            pltpu.SemaphoreType.DMA,
        ],
        compiler_params=pltpu.CompilerParams(
            use_tc_tiling_on_sc=False, needs_layout_passes=False
        ),
    )(_sc_body)
    z = sc(xv, pv)
    return z.transpose((2, 1, 3, 0)).reshape(BT_SIZE, SEQ_LEN, D_MODEL + 1)


# speedup vs baseline: 1.9087x; 1.9087x over previous
import functools

import jax
import jax.numpy as jnp
from jax import lax
from jax.experimental import pallas as pl
from jax.experimental.pallas import tpu as pltpu
from jax.experimental.pallas import tpu_sc as plsc

SEQ_LEN = 2048
BT_SIZE = 4
D_MODEL = 4096

NC = 2
NS = 16
NW = NC * NS
STC = 8
NCHUNK = (SEQ_LEN // 8) // STC
L = 16


def _sc_body(x_hbm, p_hbm, z_hbm, pos_v, ibufs, obufs, in_sems, out_sems, psem):
    wid = lax.axis_index("s") * NC + lax.axis_index("c")

    iota = lax.iota(jnp.int32, L)
    zero_v = jnp.zeros((L,), jnp.int32)
    st_pat = lax.shift_right_logical(iota, 3)
    sl_pat = lax.bitwise_and(iota, jnp.full((L, ), 7, jnp.int32))

    def start_in(c, slot):
        pltpu.make_async_copy(
            x_hbm.at[:, pl.ds(c * STC, STC), pl.ds(wid, 1), :, :],
            ibufs.at[slot, :, :, :, :, pl.ds(0, 128)],
            in_sems.at[slot],
        ).start()

    def wait_in(c, slot):
        pltpu.make_async_copy(
            x_hbm.at[:, pl.ds(c * STC, STC), pl.ds(wid, 1), :, :],
            ibufs.at[slot, :, :, :, :, pl.ds(0, 128)],
            in_sems.at[slot],
        ).wait()

    def out_copy(c, qtr):
        t0 = lax.shift_right_logical(c, 1)
        q0 = lax.bitwise_and(c, 1) * 64 + qtr * 16
        return pltpu.make_async_copy(
            obufs.at[qtr],
            z_hbm.at[pl.ds(wid * 128, 128), pl.ds(t0, 1), :, pl.ds(q0, 16)],
            out_sems.at[qtr],
        )

    def transpose_quarter(slot, qtr):
        j0 = qtr * 16
        idx_st = st_pat + (j0 // 8)
        for b in range(BT_SIZE):
            idx_b = jnp.full((L,), b, jnp.int32)
            ln0 = jnp.full((L,), 0, jnp.int32)

            def body(i, idx_ln):
                cur = idx_ln
                row = i * 8
                for u in range(8):
                    vals = plsc.load_gather(
                        ibufs.at[slot],
                        [idx_b, idx_st, zero_v, sl_pat, cur],
                    )
                    obufs[qtr, row + u, 0, b, :] = vals
                    cur = cur + 1
                return cur

            lax.fori_loop(0, 16, body, ln0)

    @pl.when(wid == NW - 1)
    def _():
        pltpu.sync_copy(p_hbm, pos_v)
        for b in range(BT_SIZE):
            pltpu.make_async_copy(
                pos_v,
                z_hbm.at[pl.ds(D_MODEL, 1), :, pl.ds(b, 1), :],
                psem,
            ).start()
        for b in range(BT_SIZE):
            pltpu.make_async_copy(
                pos_v,
                z_hbm.at[pl.ds(D_MODEL, 1), :, pl.ds(b, 1), :],
                psem,
            ).wait()

    start_in(0, 0)
    start_in(1, 1)

    def step(g, carry):
        for slot in range(2):
            c = 2 * g + slot
            wait_in(c, slot)
            for qtr in range(4):
                if slot == 1:
                    out_copy(c - 1, qtr).wait()
                else:

                    @pl.when(g > 0)
                    def _():
                        out_copy(c - 1, qtr).wait()

                transpose_quarter(slot, qtr)
                out_copy(c, qtr).start()

            @pl.when(g < NCHUNK // 2 - 1)
            def _():
                start_in(c + 2, slot)

        return carry

    lax.fori_loop(0, NCHUNK // 2, step, 0)
    for qtr in range(4):
        out_copy(NCHUNK - 1, qtr).wait()


def kernel(inputs, pos_table):
    xv = inputs.reshape(BT_SIZE, 256, 8, 32, 128).transpose(0, 1, 3, 2, 4)
    pv = pos_table.reshape(1, 16, 1, 128)
    mesh = plsc.VectorSubcoreMesh(core_axis_name="c", subcore_axis_name="s")
    sc = functools.partial(
        pl.kernel,
        mesh=mesh,
        out_type=jax.ShapeDtypeStruct((D_MODEL + 1, 16, BT_SIZE, 128), jnp.float32),
        scratch_types=[
            pltpu.VMEM((1, 16, 1, 128), jnp.float32),
            pltpu.VMEM((2, BT_SIZE, STC, 1, 8, 129), jnp.float32),
            pltpu.VMEM((4, 128, 1, BT_SIZE, 16), jnp.float32),
            pltpu.SemaphoreType.DMA((2,)),
            pltpu.SemaphoreType.DMA((4,)),
            pltpu.SemaphoreType.DMA,
        ],
        compiler_params=pltpu.CompilerParams(
            use_tc_tiling_on_sc=False, needs_layout_passes=False
        ),
    )(_sc_body)
    z = sc(xv, pv)
    return z.transpose((2, 1, 3, 0)).reshape(BT_SIZE, SEQ_LEN, D_MODEL + 1)


# speedup vs baseline: 4.0643x; 2.1294x over previous
import functools

import jax
import jax.numpy as jnp
from jax import lax
from jax.experimental import pallas as pl
from jax.experimental.pallas import tpu as pltpu
from jax.experimental.pallas import tpu_sc as plsc

SEQ_LEN = 2048
BT_SIZE = 4
D_MODEL = 4096
ROWS = SEQ_LEN * BT_SIZE

NC = 2
NS = 16
NW = NC * NS
RPW = ROWS // NW
C = 8
NCHUNK = RPW // C
L = 16


def _sc_body(x_hbm, p_hbm, o_hbm, pos_v, bufs, in_sems, out_sems):
    wid = lax.axis_index("s") * NC + lax.axis_index("c")
    base = wid * RPW
    pstart = lax.rem(base, SEQ_LEN)
    pltpu.sync_copy(p_hbm.at[pl.ds(pstart, RPW)], pos_v.at[pl.ds(0, RPW)])

    row_idx = lax.iota(jnp.int32, L)
    col_idx = jnp.full((L,), D_MODEL, jnp.int32)
    col_mask = row_idx < C

    def start_in(k, s):
        pltpu.make_async_copy(
            x_hbm.at[pl.ds(base + k * C, C), :],
            bufs.at[s, :, pl.ds(0, D_MODEL)],
            in_sems.at[s],
        ).start()

    def wait_in(k, s):
        pltpu.make_async_copy(
            x_hbm.at[pl.ds(base + k * C, C), :],
            bufs.at[s, :, pl.ds(0, D_MODEL)],
            in_sems.at[s],
        ).wait()

    def put_col(k, s):
        vals = pos_v[pl.ds(k * C, L)]
        plsc.store_scatter(bufs.at[s], [row_idx, col_idx], vals, mask=col_mask)

    def out_copy(k, s):
        return pltpu.make_async_copy(
            bufs.at[s],
            o_hbm.at[pl.ds(base + k * C, C), :],
            out_sems.at[s],
        )

    start_in(0, 0)
    start_in(1, 1)

    def step(g, carry):
        k0 = 2 * g
        put_col(k0, 0)
        wait_in(k0, 0)
        out_copy(k0, 0).start()
        put_col(k0 + 1, 1)
        wait_in(k0 + 1, 1)
        out_copy(k0 + 1, 1).start()
        out_copy(k0, 0).wait()
        start_in(k0 + 2, 0)
        out_copy(k0 + 1, 1).wait()
        start_in(k0 + 3, 1)
        return carry

    lax.fori_loop(0, NCHUNK // 2 - 1, step, 0)

    kl = NCHUNK - 2
    put_col(kl, 0)
    wait_in(kl, 0)
    out_copy(kl, 0).start()
    put_col(kl + 1, 1)
    wait_in(kl + 1, 1)
    out_copy(kl + 1, 1).start()
    out_copy(kl, 0).wait()
    out_copy(kl + 1, 1).wait()


def kernel(inputs, pos_table):
    x = inputs.reshape(ROWS, D_MODEL)
    p = pos_table.reshape(SEQ_LEN)
    mesh = plsc.VectorSubcoreMesh(core_axis_name="c", subcore_axis_name="s")
    sc_copy = functools.partial(
        pl.kernel,
        mesh=mesh,
        out_type=jax.ShapeDtypeStruct((ROWS, D_MODEL + 1), jnp.float32),
        scratch_types=[
            pltpu.VMEM((RPW + L,), jnp.float32),
            pltpu.VMEM((2, C, D_MODEL + 1), jnp.float32),
            pltpu.SemaphoreType.DMA((2,)),
            pltpu.SemaphoreType.DMA((2,)),
        ],
        compiler_params=pltpu.CompilerParams(needs_layout_passes=False),
    )(_sc_body)
    out = sc_copy(x, p)
    return out.reshape(BT_SIZE, SEQ_LEN, D_MODEL + 1)
